# Initial kernel scaffold; baseline (speedup 1.0000x reference)
import jax
import jax.numpy as jnp
from jax.experimental import pallas as pl


def kernel(logits, gt_tokens, node_types, mask_positions):
    raise NotImplementedError("write your pallas kernel here")


# speedup vs baseline: 1793.9740x; 1793.9740x over previous
import functools

import jax
import jax.numpy as jnp
from jax import lax
from jax.experimental import pallas as pl
from jax.experimental.pallas import tpu as pltpu
from jax.experimental.pallas import tpu_sc as plsc

B, N, V = 8, 512, 8192
T = 4
NG = T * B
NC, NS, L = 2, 16, 16

BIGF = 1e30
CACHE_ROWS = 224
COST_WORDS = (CACHE_ROWS + 1) * N


_BV = 1024
_NVB = V // _BV


def _lse_body(logits_ref, maskT_ref, sum_ref, cnt_ref, m_ref, s_ref):
    bstep = pl.program_id(0)
    vstep = pl.program_id(1)

    @pl.when(vstep == 0)
    def _():
        m_ref[...] = jnp.full((N, 1), -BIGF, jnp.float32)
        s_ref[...] = jnp.zeros((N, 1), jnp.float32)

    blk = logits_ref[0]
    m_old = m_ref[...]
    m_new = jnp.maximum(m_old, jnp.max(blk, axis=1, keepdims=True))
    s_ref[...] = (s_ref[...] * jnp.exp(m_old - m_new)
                  + jnp.sum(jnp.exp(blk - m_new), axis=1, keepdims=True))
    m_ref[...] = m_new

    @pl.when((vstep == 0) & (bstep == 0))
    def _():
        sum_ref[...] = jnp.zeros((B, 128), jnp.float32)
        cnt_ref[...] = jnp.zeros((B, 128), jnp.float32)

    @pl.when(vstep == _NVB - 1)
    def _():
        lse = m_ref[...] + jnp.log(s_ref[...])
        onehot = (lax.broadcasted_iota(jnp.int32, (N, B), 1) == bstep)
        mk = jnp.sum(jnp.where(onehot, maskT_ref[...], 0.0),
                     axis=1, keepdims=True)
        rowsel = lax.broadcasted_iota(jnp.int32, (B, 128), 0) == bstep
        sum_ref[...] = jnp.where(rowsel, jnp.sum(lse * mk), sum_ref[...])
        cnt_ref[...] = jnp.where(rowsel, jnp.sum(mk), cnt_ref[...])


def _masked_lse(logits, mask_f):
    maskT = mask_f.T.reshape(N, B)
    sums, cnts = pl.pallas_call(
        _lse_body,
        grid=(B, _NVB),
        in_specs=[
            pl.BlockSpec((1, N, _BV), lambda b, v: (b, 0, v)),
            pl.BlockSpec((N, B), lambda b, v: (0, 0)),
        ],
        out_specs=[
            pl.BlockSpec((B, 128), lambda b, v: (0, 0)),
            pl.BlockSpec((B, 128), lambda b, v: (0, 0)),
        ],
        out_shape=[
            jax.ShapeDtypeStruct((B, 128), jnp.float32),
            jax.ShapeDtypeStruct((B, 128), jnp.float32),
        ],
        scratch_shapes=[
            pltpu.VMEM((N, 1), jnp.float32),
            pltpu.VMEM((N, 1), jnp.float32),
        ],
    )(logits, maskT)
    return jnp.sum(sums[:, 0]), jnp.sum(cnts[:, 0])


def _iota16():
    return lax.iota(jnp.int32, L)


def _scalar(x):
    return jnp.max(x)


def _vread(ref, i):
    return jnp.max(plsc.load_gather(ref, [jnp.full((L,), i, jnp.int32)]))


def _vwrite(ref, i, val):
    plsc.store_scatter(
        ref,
        [jnp.full((L,), i, jnp.int32)],
        jnp.full((L,), val, ref.dtype),
        mask=_iota16() == 0,
    )


def _sc_body(logits_hbm, gt_hbm, types_hbm, mask_hbm, out_hbm,
             gt_v, types_v, mask_v, pos_v, gts_v,
             cost_v, u_v, v_v, minv_v, validf_v, way_v, used_v, intree_v,
             idx_v, out_v, pcol_s, sem):
    wid = lax.axis_index("s") * NC + lax.axis_index("c")
    t = wid // B
    b = wid % B

    pltpu.sync_copy(gt_hbm.at[b], gt_v)
    pltpu.sync_copy(types_hbm.at[b], types_v)
    pltpu.sync_copy(mask_hbm.at[b], mask_v)

    def comp_chunk(c, kcur):
        off = pl.multiple_of(c * L, L)
        tv = types_v[pl.ds(off, L)]
        mv = mask_v[pl.ds(off, L)]
        gv = gt_v[pl.ds(off, L)]
        sel = (tv == t) & (mv != 0)
        cs = plsc.cumsum(sel.astype(jnp.int32))
        dest = kcur + cs - 1
        posv = off + _iota16()
        plsc.store_scatter(pos_v, [dest], posv, mask=sel)
        plsc.store_scatter(gts_v, [dest], gv, mask=sel)
        return kcur + jnp.max(cs)

    k = lax.fori_loop(0, N // L, comp_chunk, jnp.int32(0))
    nv = (k + L - 1) // L
    nch = (k + 127) // 128

    out_v[...] = jnp.zeros((L,), jnp.float32)

    def gather_row(i0, dst_row):
        base = (b * N + _vread(pos_v, i0)) * V

        def g_chunk(cc, _):
            for s8 in range(128 // L):
                j0 = cc * 128 + s8 * L
                jv = j0 + _iota16()
                gsub = gts_v[pl.ds(pl.multiple_of(j0, L), L)]
                gsub = jnp.where(jv < k, gsub, 0)
                idx_v[pl.ds(s8 * L, L)] = base + gsub
            dst = cost_v.at[pl.ds(pl.multiple_of(dst_row * N + cc * 128, 128), 128)]
            pltpu.async_copy(logits_hbm.at[idx_v], dst, sem).wait()
            return 0

        lax.fori_loop(0, nch, g_chunk, 0)

    @pl.when(k > 0)
    def _solve():
        def prefetch(i0, _):
            gather_row(i0, i0)
            return 0

        lax.fori_loop(0, jnp.minimum(k, CACHE_ROWS), prefetch, 0)

        def init_uv(c, _):
            off = pl.multiple_of(c * L, L)
            z = jnp.zeros((L,), jnp.float32)
            u_v[pl.ds(off, L)] = z
            v_v[pl.ds(off, L)] = z
            validf_v[pl.ds(off, L)] = jnp.where(off + _iota16() < k, 1.0, 0.0
                                                ).astype(jnp.float32)
            return 0

        lax.fori_loop(0, nv, init_uv, 0)

        def init_pcol(i, _):
            pcol_s[i] = jnp.int32(-1)
            return 0

        lax.fori_loop(0, k, init_pcol, 0)

        def phase(r, _):
            def init_chunk(c, _):
                off = pl.multiple_of(c * L, L)
                minv_v[pl.ds(off, L)] = jnp.full((L,), BIGF, jnp.float32)
                way_v[pl.ds(off, L)] = jnp.full((L,), -2, jnp.int32)
                z = jnp.zeros((L,), jnp.float32)
                used_v[pl.ds(off, L)] = z
                intree_v[pl.ds(off, L)] = z
                return 0

            lax.fori_loop(0, nv, init_chunk, 0)

            def inner_cond(carry):
                return carry[2] != 0

            def inner_body(carry):
                jc, i0, _go = carry
                _vwrite(intree_v, i0, 1.0)

                lax.cond(i0 < CACHE_ROWS,
                         lambda: None,
                         lambda: gather_row(i0, CACHE_ROWS))
                ro = pl.multiple_of(jnp.minimum(i0, CACHE_ROWS) * N, 128)
                ubase = _vread(u_v, i0)

                def scan_chunk(c, carry2):
                    bestv, bestc = carry2
                    off = pl.multiple_of(c * L, L)
                    cu = (-cost_v[pl.ds(pl.multiple_of(ro + off, L), L)]
                          - ubase - v_v[pl.ds(off, L)])
                    usedc = used_v[pl.ds(off, L)]
                    unused = usedc < 0.5
                    mv = minv_v[pl.ds(off, L)]
                    upd = unused & (cu < mv)
                    mv2 = jnp.where(upd, cu, mv)
                    minv_v[pl.ds(off, L)] = mv2
                    wc = way_v[pl.ds(off, L)]
                    way_v[pl.ds(off, L)] = jnp.where(upd, jc, wc)
                    elig = unused & (validf_v[pl.ds(off, L)] > 0.5)
                    masked = jnp.where(elig, mv2, BIGF)
                    cmin = jnp.min(masked)
                    better = cmin < bestv
                    return (jnp.where(better, cmin, bestv),
                            jnp.where(better, c, bestc))

                delta, bestc = lax.fori_loop(0, nv, scan_chunk,
                                             (jnp.float32(BIGF), jnp.int32(0)))

                off = pl.multiple_of(bestc * L, L)
                mv2 = minv_v[pl.ds(off, L)]
                elig = ((used_v[pl.ds(off, L)] < 0.5)
                        & (validf_v[pl.ds(off, L)] > 0.5))
                masked = jnp.where(elig, mv2, BIGF)
                lane = _scalar(plsc.all_reduce_ffs(masked <= delta))
                j1 = bestc * L + lane

                def upd_chunk(c, _):
                    off = pl.multiple_of(c * L, L)
                    it = intree_v[pl.ds(off, L)]
                    u_v[pl.ds(off, L)] = u_v[pl.ds(off, L)] + delta * it
                    us = used_v[pl.ds(off, L)]
                    v_v[pl.ds(off, L)] = v_v[pl.ds(off, L)] - delta * us
                    minv_v[pl.ds(off, L)] = (minv_v[pl.ds(off, L)]
                                             - delta * (1.0 - us))
                    return 0

                lax.fori_loop(0, nv, upd_chunk, 0)
                _vwrite(used_v, j1, 1.0)
                i0n = pcol_s[j1]
                return (j1, i0n, (i0n >= 0).astype(jnp.int32))

            jcf, _, _ = lax.while_loop(inner_cond, inner_body,
                                       (jnp.int32(-1), r.astype(jnp.int32),
                                        jnp.int32(1)))

            def aug_body(carry):
                j, _go = carry
                prev = _vread(way_v, j)
                pv = pcol_s[jnp.maximum(prev, 0)]
                pcol_s[j] = jnp.where(prev >= 0, pv, r.astype(jnp.int32))
                return (jnp.maximum(prev, 0), (prev >= 0).astype(jnp.int32))

            lax.while_loop(lambda c: c[1] != 0, aug_body, (jcf, jnp.int32(1)))
            return 0

        lax.fori_loop(0, k, phase, 0)

        def sum_chunk(c, acc):
            off = pl.multiple_of(c * L, L)
            return acc + jnp.sum(u_v[pl.ds(off, L)]) + jnp.sum(v_v[pl.ds(off, L)])

        val = lax.fori_loop(0, nv, sum_chunk, jnp.float32(0.0))
        out_v[...] = jnp.full((L,), val, jnp.float32)

    pltpu.sync_copy(out_v, out_hbm.at[wid])


def _sc_matching(logits_flat, gt, types, mask_i):
    mesh = plsc.VectorSubcoreMesh(core_axis_name="c", subcore_axis_name="s")
    f = pl.kernel(
        _sc_body,
        out_type=jax.ShapeDtypeStruct((NG, L), jnp.float32),
        mesh=mesh,
        compiler_params=pltpu.CompilerParams(needs_layout_passes=False),
        scratch_types=[
            pltpu.VMEM((N,), jnp.int32),
            pltpu.VMEM((N,), jnp.int32),
            pltpu.VMEM((N,), jnp.int32),
            pltpu.VMEM((N,), jnp.int32),
            pltpu.VMEM((N,), jnp.int32),
            pltpu.VMEM((COST_WORDS,), jnp.float32),
            pltpu.VMEM((N,), jnp.float32),
            pltpu.VMEM((N,), jnp.float32),
            pltpu.VMEM((N,), jnp.float32),
            pltpu.VMEM((N,), jnp.float32),
            pltpu.VMEM((N,), jnp.int32),
            pltpu.VMEM((N,), jnp.float32),
            pltpu.VMEM((N,), jnp.float32),
            pltpu.VMEM((128,), jnp.int32),
            pltpu.VMEM((L,), jnp.float32),
            pltpu.SMEM((N,), jnp.int32),
            pltpu.SemaphoreType.DMA,
        ],
    )
    return f(logits_flat, gt, types, mask_i)


def kernel(logits, gt_tokens, node_types, mask_positions):
    logits = logits.astype(jnp.float32)
    gt = gt_tokens.astype(jnp.int32)
    types = node_types.astype(jnp.int32)
    mask_i = mask_positions.astype(jnp.int32)

    lse_sum, cnt = _masked_lse(logits, mask_positions.astype(jnp.float32))
    duals = _sc_matching(logits.reshape(-1), gt, types, mask_i)
    dual_sum = jnp.sum(duals[:, 0])
    loss = (lse_sum + dual_sum) / cnt
    return jnp.where(cnt > 0, loss, jnp.zeros((), jnp.float32))


# speedup vs baseline: 5710.3010x; 3.1830x over previous
import functools

import jax
import jax.numpy as jnp
from jax import lax
from jax.experimental import pallas as pl
from jax.experimental.pallas import tpu as pltpu
from jax.experimental.pallas import tpu_sc as plsc

B, N, V = 8, 512, 8192
T = 4
NG = T * B
NC, NS, L = 2, 16, 16

BIGF = 1e30
CACHE_ROWS = 224
COST_WORDS = (CACHE_ROWS + 1) * N


_BV = 1024
_NVB = V // _BV


def _lse_body(logits_ref, maskT_ref, sum_ref, cnt_ref, m_ref, s_ref):
    bstep = pl.program_id(0)
    vstep = pl.program_id(1)

    @pl.when(vstep == 0)
    def _():
        m_ref[...] = jnp.full((N, 1), -BIGF, jnp.float32)
        s_ref[...] = jnp.zeros((N, 1), jnp.float32)

    blk = logits_ref[0]
    m_old = m_ref[...]
    m_new = jnp.maximum(m_old, jnp.max(blk, axis=1, keepdims=True))
    s_ref[...] = (s_ref[...] * jnp.exp(m_old - m_new)
                  + jnp.sum(jnp.exp(blk - m_new), axis=1, keepdims=True))
    m_ref[...] = m_new

    @pl.when((vstep == 0) & (bstep == 0))
    def _():
        sum_ref[...] = jnp.zeros((B, 128), jnp.float32)
        cnt_ref[...] = jnp.zeros((B, 128), jnp.float32)

    @pl.when(vstep == _NVB - 1)
    def _():
        lse = m_ref[...] + jnp.log(s_ref[...])
        onehot = (lax.broadcasted_iota(jnp.int32, (N, B), 1) == bstep)
        mk = jnp.sum(jnp.where(onehot, maskT_ref[...], 0.0),
                     axis=1, keepdims=True)
        rowsel = lax.broadcasted_iota(jnp.int32, (B, 128), 0) == bstep
        sum_ref[...] = jnp.where(rowsel, jnp.sum(lse * mk), sum_ref[...])
        cnt_ref[...] = jnp.where(rowsel, jnp.sum(mk), cnt_ref[...])


def _masked_lse(logits, mask_f):
    maskT = mask_f.T.reshape(N, B)
    sums, cnts = pl.pallas_call(
        _lse_body,
        grid=(B, _NVB),
        in_specs=[
            pl.BlockSpec((1, N, _BV), lambda b, v: (b, 0, v)),
            pl.BlockSpec((N, B), lambda b, v: (0, 0)),
        ],
        out_specs=[
            pl.BlockSpec((B, 128), lambda b, v: (0, 0)),
            pl.BlockSpec((B, 128), lambda b, v: (0, 0)),
        ],
        out_shape=[
            jax.ShapeDtypeStruct((B, 128), jnp.float32),
            jax.ShapeDtypeStruct((B, 128), jnp.float32),
        ],
        scratch_shapes=[
            pltpu.VMEM((N, 1), jnp.float32),
            pltpu.VMEM((N, 1), jnp.float32),
        ],
    )(logits, maskT)
    return jnp.sum(sums[:, 0]), jnp.sum(cnts[:, 0])


def _iota16():
    return lax.iota(jnp.int32, L)


def _scalar(x):
    return jnp.max(x)


def _vread(ref, i):
    return jnp.max(plsc.load_gather(ref, [jnp.full((L,), i, jnp.int32)]))


def _vwrite(ref, i, val):
    plsc.store_scatter(
        ref,
        [jnp.full((L,), i, jnp.int32)],
        jnp.full((L,), val, ref.dtype),
        mask=_iota16() == 0,
    )


def _sc_body(logits_hbm, gt_hbm, types_hbm, mask_hbm, out_hbm,
             gt_v, types_v, mask_v, pos_v, gts_v,
             cost_v, u_v, v_v, minv_v, validf_v, way_v, used_v, intree_v,
             colmin_v, idx_v, out_v, pcol_s, sem):
    wid = lax.axis_index("s") * NC + lax.axis_index("c")
    t = wid // B
    b = wid % B

    pltpu.sync_copy(gt_hbm.at[b], gt_v)
    pltpu.sync_copy(types_hbm.at[b], types_v)
    pltpu.sync_copy(mask_hbm.at[b], mask_v)

    def comp_chunk(c, kcur):
        off = pl.multiple_of(c * L, L)
        tv = types_v[pl.ds(off, L)]
        mv = mask_v[pl.ds(off, L)]
        gv = gt_v[pl.ds(off, L)]
        sel = (tv == t) & (mv != 0)
        cs = plsc.cumsum(sel.astype(jnp.int32))
        dest = kcur + cs - 1
        posv = off + _iota16()
        plsc.store_scatter(pos_v, [dest], posv, mask=sel)
        plsc.store_scatter(gts_v, [dest], gv, mask=sel)
        return kcur + jnp.max(cs)

    k = lax.fori_loop(0, N // L, comp_chunk, jnp.int32(0))
    nv = (k + L - 1) // L
    nch = (k + 127) // 128

    out_v[...] = jnp.zeros((L,), jnp.float32)

    def gather_row(i0, dst_row):
        base = (b * N + _vread(pos_v, i0)) * V

        def g_chunk(cc, _):
            for s8 in range(128 // L):
                j0 = cc * 128 + s8 * L
                jv = j0 + _iota16()
                gsub = gts_v[pl.ds(pl.multiple_of(j0, L), L)]
                gsub = jnp.where(jv < k, gsub, 0)
                idx_v[pl.ds(s8 * L, L)] = base + gsub
            dst = cost_v.at[pl.ds(pl.multiple_of(dst_row * N + cc * 128, 128), 128)]
            pltpu.async_copy(logits_hbm.at[idx_v], dst, sem).wait()
            return 0

        lax.fori_loop(0, nch, g_chunk, 0)

    @pl.when(k > 0)
    def _solve():
        def init_cr(c, _):
            off = pl.multiple_of(c * L, L)
            colmin_v[pl.ds(off, L)] = jnp.full((L,), BIGF, jnp.float32)
            return 0

        lax.fori_loop(0, nv, init_cr, 0)

        def prefetch(i0, _):
            dst = jnp.minimum(i0, CACHE_ROWS)
            gather_row(i0, dst)
            ro = pl.multiple_of(dst * N, 128)

            def cr_chunk(c, _):
                off = pl.multiple_of(c * L, L)
                cu = -cost_v[pl.ds(pl.multiple_of(ro + off, L), L)]
                cm = colmin_v[pl.ds(off, L)]
                colmin_v[pl.ds(off, L)] = jnp.where(cu < cm, cu, cm)
                return 0

            lax.fori_loop(0, nv, cr_chunk, 0)
            return 0

        lax.fori_loop(0, k, prefetch, 0)

        def init_uv(c, _):
            off = pl.multiple_of(c * L, L)
            z = jnp.zeros((L,), jnp.float32)
            u_v[pl.ds(off, L)] = z
            valid = jnp.where(off + _iota16() < k, 1.0, 0.0).astype(jnp.float32)
            validf_v[pl.ds(off, L)] = valid
            v_v[pl.ds(off, L)] = jnp.where(valid > 0.5,
                                           colmin_v[pl.ds(off, L)], 0.0)
            return 0

        lax.fori_loop(0, nv, init_uv, 0)

        def init_pcol(i, _):
            pcol_s[i] = jnp.int32(-1)
            return 0

        lax.fori_loop(0, k, init_pcol, 0)

        def phase(r, _):
            def init_chunk(c, _):
                off = pl.multiple_of(c * L, L)
                minv_v[pl.ds(off, L)] = jnp.full((L,), BIGF, jnp.float32)
                way_v[pl.ds(off, L)] = jnp.full((L,), -2, jnp.int32)
                z = jnp.zeros((L,), jnp.float32)
                used_v[pl.ds(off, L)] = z
                intree_v[pl.ds(off, L)] = z
                return 0

            lax.fori_loop(0, nv, init_chunk, 0)

            def inner_cond(carry):
                return carry[2] != 0

            def inner_body(carry):
                jc, i0, _go = carry
                _vwrite(intree_v, i0, 1.0)

                lax.cond(i0 < CACHE_ROWS,
                         lambda: None,
                         lambda: gather_row(i0, CACHE_ROWS))
                ro = pl.multiple_of(jnp.minimum(i0, CACHE_ROWS) * N, 128)
                ubase = _vread(u_v, i0)

                def scan_chunk(c, carry2):
                    bestv, bestc = carry2
                    off = pl.multiple_of(c * L, L)
                    cu = (-cost_v[pl.ds(pl.multiple_of(ro + off, L), L)]
                          - ubase - v_v[pl.ds(off, L)])
                    usedc = used_v[pl.ds(off, L)]
                    unused = usedc < 0.5
                    mv = minv_v[pl.ds(off, L)]
                    upd = unused & (cu < mv)
                    mv2 = jnp.where(upd, cu, mv)
                    minv_v[pl.ds(off, L)] = mv2
                    wc = way_v[pl.ds(off, L)]
                    way_v[pl.ds(off, L)] = jnp.where(upd, jc, wc)
                    elig = unused & (validf_v[pl.ds(off, L)] > 0.5)
                    masked = jnp.where(elig, mv2, BIGF)
                    cmin = jnp.min(masked)
                    better = cmin < bestv
                    return (jnp.where(better, cmin, bestv),
                            jnp.where(better, c, bestc))

                delta, bestc = lax.fori_loop(0, nv, scan_chunk,
                                             (jnp.float32(BIGF), jnp.int32(0)))

                off = pl.multiple_of(bestc * L, L)
                mv2 = minv_v[pl.ds(off, L)]
                elig = ((used_v[pl.ds(off, L)] < 0.5)
                        & (validf_v[pl.ds(off, L)] > 0.5))
                masked = jnp.where(elig, mv2, BIGF)
                lane = _scalar(plsc.all_reduce_ffs(masked <= delta))
                j1 = bestc * L + lane

                def upd_chunk(c, _):
                    off = pl.multiple_of(c * L, L)
                    it = intree_v[pl.ds(off, L)]
                    u_v[pl.ds(off, L)] = u_v[pl.ds(off, L)] + delta * it
                    us = used_v[pl.ds(off, L)]
                    v_v[pl.ds(off, L)] = v_v[pl.ds(off, L)] - delta * us
                    minv_v[pl.ds(off, L)] = (minv_v[pl.ds(off, L)]
                                             - delta * (1.0 - us))
                    return 0

                lax.fori_loop(0, nv, upd_chunk, 0)
                _vwrite(used_v, j1, 1.0)
                i0n = pcol_s[j1]
                return (j1, i0n, (i0n >= 0).astype(jnp.int32))

            jcf, _, _ = lax.while_loop(inner_cond, inner_body,
                                       (jnp.int32(-1), r.astype(jnp.int32),
                                        jnp.int32(1)))

            def aug_body(carry):
                j, _go = carry
                prev = _vread(way_v, j)
                pv = pcol_s[jnp.maximum(prev, 0)]
                pcol_s[j] = jnp.where(prev >= 0, pv, r.astype(jnp.int32))
                return (jnp.maximum(prev, 0), (prev >= 0).astype(jnp.int32))

            lax.while_loop(lambda c: c[1] != 0, aug_body, (jcf, jnp.int32(1)))
            return 0

        lax.fori_loop(0, k, phase, 0)

        def sum_chunk(c, acc):
            off = pl.multiple_of(c * L, L)
            return acc + jnp.sum(u_v[pl.ds(off, L)]) + jnp.sum(v_v[pl.ds(off, L)])

        val = lax.fori_loop(0, nv, sum_chunk, jnp.float32(0.0))
        out_v[...] = jnp.full((L,), val, jnp.float32)

    pltpu.sync_copy(out_v, out_hbm.at[wid])


def _sc_matching(logits_flat, gt, types, mask_i):
    mesh = plsc.VectorSubcoreMesh(core_axis_name="c", subcore_axis_name="s")
    f = pl.kernel(
        _sc_body,
        out_type=jax.ShapeDtypeStruct((NG, L), jnp.float32),
        mesh=mesh,
        compiler_params=pltpu.CompilerParams(needs_layout_passes=False),
        scratch_types=[
            pltpu.VMEM((N,), jnp.int32),
            pltpu.VMEM((N,), jnp.int32),
            pltpu.VMEM((N,), jnp.int32),
            pltpu.VMEM((N,), jnp.int32),
            pltpu.VMEM((N,), jnp.int32),
            pltpu.VMEM((COST_WORDS,), jnp.float32),
            pltpu.VMEM((N,), jnp.float32),
            pltpu.VMEM((N,), jnp.float32),
            pltpu.VMEM((N,), jnp.float32),
            pltpu.VMEM((N,), jnp.float32),
            pltpu.VMEM((N,), jnp.int32),
            pltpu.VMEM((N,), jnp.float32),
            pltpu.VMEM((N,), jnp.float32),
            pltpu.VMEM((N,), jnp.float32),
            pltpu.VMEM((128,), jnp.int32),
            pltpu.VMEM((L,), jnp.float32),
            pltpu.SMEM((N,), jnp.int32),
            pltpu.SemaphoreType.DMA,
        ],
    )
    return f(logits_flat, gt, types, mask_i)


def kernel(logits, gt_tokens, node_types, mask_positions):
    logits = logits.astype(jnp.float32)
    gt = gt_tokens.astype(jnp.int32)
    types = node_types.astype(jnp.int32)
    mask_i = mask_positions.astype(jnp.int32)

    lse_sum, cnt = _masked_lse(logits, mask_positions.astype(jnp.float32))
    duals = _sc_matching(logits.reshape(-1), gt, types, mask_i)
    dual_sum = jnp.sum(duals[:, 0])
    loss = (lse_sum + dual_sum) / cnt
    return jnp.where(cnt > 0, loss, jnp.zeros((), jnp.float32))


# speedup vs baseline: 7058.5067x; 1.2361x over previous
import functools

import jax
import jax.numpy as jnp
from jax import lax
from jax.experimental import pallas as pl
from jax.experimental.pallas import tpu as pltpu
from jax.experimental.pallas import tpu_sc as plsc

B, N, V = 8, 512, 8192
T = 4
NG = T * B
NC, NS, L = 2, 16, 16

BIGF = 1e30
CAP_SLABS = 230
CAP_WORDS = CAP_SLABS * 512
OVER_SLAB = CAP_SLABS - 1
OVER_OFF = OVER_SLAB * 512


_BV = 1024
_NVB = V // _BV


def _lse_body(logits_ref, maskT_ref, sum_ref, cnt_ref, m_ref, s_ref):
    bstep = pl.program_id(0)
    vstep = pl.program_id(1)

    @pl.when(vstep == 0)
    def _():
        m_ref[...] = jnp.full((N, 1), -BIGF, jnp.float32)
        s_ref[...] = jnp.zeros((N, 1), jnp.float32)

    blk = logits_ref[0]
    m_old = m_ref[...]
    m_new = jnp.maximum(m_old, jnp.max(blk, axis=1, keepdims=True))
    s_ref[...] = (s_ref[...] * jnp.exp(m_old - m_new)
                  + jnp.sum(jnp.exp(blk - m_new), axis=1, keepdims=True))
    m_ref[...] = m_new

    @pl.when((vstep == 0) & (bstep == 0))
    def _():
        sum_ref[...] = jnp.zeros((B, 128), jnp.float32)
        cnt_ref[...] = jnp.zeros((B, 128), jnp.float32)

    @pl.when(vstep == _NVB - 1)
    def _():
        lse = m_ref[...] + jnp.log(s_ref[...])
        onehot = (lax.broadcasted_iota(jnp.int32, (N, B), 1) == bstep)
        mk = jnp.sum(jnp.where(onehot, maskT_ref[...], 0.0),
                     axis=1, keepdims=True)
        rowsel = lax.broadcasted_iota(jnp.int32, (B, 128), 0) == bstep
        sum_ref[...] = jnp.where(rowsel, jnp.sum(lse * mk), sum_ref[...])
        cnt_ref[...] = jnp.where(rowsel, jnp.sum(mk), cnt_ref[...])


def _masked_lse(logits, mask_f):
    maskT = mask_f.T.reshape(N, B)
    sums, cnts = pl.pallas_call(
        _lse_body,
        grid=(B, _NVB),
        in_specs=[
            pl.BlockSpec((1, N, _BV), lambda b, v: (b, 0, v)),
            pl.BlockSpec((N, B), lambda b, v: (0, 0)),
        ],
        out_specs=[
            pl.BlockSpec((B, 128), lambda b, v: (0, 0)),
            pl.BlockSpec((B, 128), lambda b, v: (0, 0)),
        ],
        out_shape=[
            jax.ShapeDtypeStruct((B, 128), jnp.float32),
            jax.ShapeDtypeStruct((B, 128), jnp.float32),
        ],
        scratch_shapes=[
            pltpu.VMEM((N, 1), jnp.float32),
            pltpu.VMEM((N, 1), jnp.float32),
        ],
    )(logits, maskT)
    return jnp.sum(sums[:, 0]), jnp.sum(cnts[:, 0])


def _iota16():
    return lax.iota(jnp.int32, L)


def _scalar(x):
    return jnp.max(x)


def _vread(ref, i):
    return jnp.max(plsc.load_gather(ref, [jnp.full((L,), i, jnp.int32)]))


def _vwrite(ref, i, val):
    plsc.store_scatter(
        ref,
        [jnp.full((L,), i, jnp.int32)],
        jnp.full((L,), val, ref.dtype),
        mask=_iota16() == 0,
    )


def _sc_body(logits_hbm, gt_hbm, types_hbm, mask_hbm, out_hbm,
             gt_v, types_v, mask_v, pos_v, gts_v, base_v,
             cost_v, u_v, v_v, minv_v, validf_v, way_v, used_v, intree_v,
             colmin_v, idx_v, out_v, pcol_s, sem):
    wid = lax.axis_index("s") * NC + lax.axis_index("c")
    t = wid // B
    b = wid % B

    pltpu.sync_copy(gt_hbm.at[b], gt_v)
    pltpu.sync_copy(types_hbm.at[b], types_v)
    pltpu.sync_copy(mask_hbm.at[b], mask_v)

    def comp_chunk(c, kcur):
        off = pl.multiple_of(c * L, L)
        tv = types_v[pl.ds(off, L)]
        mv = mask_v[pl.ds(off, L)]
        gv = gt_v[pl.ds(off, L)]
        sel = (tv == t) & (mv != 0)
        cs = plsc.cumsum(sel.astype(jnp.int32))
        dest = kcur + cs - 1
        posv = off + _iota16()
        plsc.store_scatter(pos_v, [dest], posv, mask=sel)
        plsc.store_scatter(gts_v, [dest], gv, mask=sel)
        return kcur + jnp.max(cs)

    k = lax.fori_loop(0, N // L, comp_chunk, jnp.int32(0))
    nv = (k + L - 1) // L
    ks = nv * L
    rows_cached = jnp.minimum(k, OVER_OFF // jnp.maximum(ks, 1))
    m2 = (rows_cached * ks + 511) // 512

    out_v[...] = jnp.zeros((L,), jnp.float32)

    def _chunk_ld(w):
        return cost_v[pl.ds(pl.multiple_of(w, L), L)]

    def _sub_idx(i, c):
        base = _vread(base_v, jnp.minimum(i, k - 1))
        j0 = pl.multiple_of(c * L, L)
        g = gts_v[pl.ds(j0, L)]
        g = jnp.where(j0 + _iota16() < k, g, 0)
        return base + g

    def gather_slab(m):
        def bsub(sl, _):
            sg = m * 32 + sl
            q = sg // nv
            c = sg - q * nv
            idx_v[pl.ds(pl.multiple_of(sl * L, L), L)] = _sub_idx(q, c)
            return 0

        lax.fori_loop(0, 32, bsub, 0)
        dst = cost_v.at[pl.ds(pl.multiple_of(m * 512, 512), 512)]
        pltpu.async_copy(logits_hbm.at[idx_v], dst, sem).wait()

    def gather_row_over(i0):
        def bsub(sl, _):
            c = jnp.minimum(sl, nv - 1)
            idx_v[pl.ds(pl.multiple_of(sl * L, L), L)] = _sub_idx(i0, c)
            return 0

        lax.fori_loop(0, 32, bsub, 0)
        dst = cost_v.at[pl.ds(OVER_OFF, 512)]
        pltpu.async_copy(logits_hbm.at[idx_v], dst, sem).wait()

    @pl.when(k > 0)
    def _solve():
        def baseb(c, _):
            off = pl.multiple_of(c * L, L)
            base_v[pl.ds(off, L)] = (b * N + pos_v[pl.ds(off, L)]) * V
            return 0

        lax.fori_loop(0, nv, baseb, 0)

        def g_loop(m, _):
            gather_slab(m)
            return 0

        lax.fori_loop(0, m2, g_loop, 0)

        def init_cr(c, _):
            off = pl.multiple_of(c * L, L)
            colmin_v[pl.ds(off, L)] = jnp.full((L,), BIGF, jnp.float32)
            return 0

        lax.fori_loop(0, nv, init_cr, 0)

        def cr_row(i0, _):
            ro = i0 * ks

            def cr_chunk(c, _):
                off = pl.multiple_of(c * L, L)
                cu = -_chunk_ld(ro + off)
                cm = colmin_v[pl.ds(off, L)]
                colmin_v[pl.ds(off, L)] = jnp.where(cu < cm, cu, cm)
                return 0

            lax.fori_loop(0, nv, cr_chunk, 0)
            return 0

        lax.fori_loop(0, rows_cached, cr_row, 0)

        def cr_tail(i0, _):
            gather_row_over(i0)

            def cr_chunk(c, _):
                off = pl.multiple_of(c * L, L)
                cu = -_chunk_ld(OVER_OFF + off)
                cm = colmin_v[pl.ds(off, L)]
                colmin_v[pl.ds(off, L)] = jnp.where(cu < cm, cu, cm)
                return 0

            lax.fori_loop(0, nv, cr_chunk, 0)
            return 0

        lax.fori_loop(rows_cached, k, cr_tail, 0)

        def init_uv(c, _):
            off = pl.multiple_of(c * L, L)
            z = jnp.zeros((L,), jnp.float32)
            u_v[pl.ds(off, L)] = z
            valid = jnp.where(off + _iota16() < k, 1.0, 0.0).astype(jnp.float32)
            validf_v[pl.ds(off, L)] = valid
            v_v[pl.ds(off, L)] = jnp.where(valid > 0.5,
                                           colmin_v[pl.ds(off, L)], 0.0)
            return 0

        lax.fori_loop(0, nv, init_uv, 0)

        def init_pcol(i, _):
            pcol_s[i] = jnp.int32(-1)
            return 0

        lax.fori_loop(0, k, init_pcol, 0)

        def phase(r, _):
            def init_chunk(c, _):
                off = pl.multiple_of(c * L, L)
                minv_v[pl.ds(off, L)] = jnp.full((L,), BIGF, jnp.float32)
                way_v[pl.ds(off, L)] = jnp.full((L,), -2, jnp.int32)
                z = jnp.zeros((L,), jnp.float32)
                used_v[pl.ds(off, L)] = z
                intree_v[pl.ds(off, L)] = z
                return 0

            lax.fori_loop(0, nv, init_chunk, 0)

            def inner_cond(carry):
                return carry[2] != 0

            def inner_body(carry):
                jc, i0, _go = carry
                _vwrite(intree_v, i0, 1.0)

                lax.cond(i0 < rows_cached,
                         lambda: None,
                         lambda: gather_row_over(i0))
                ro = jnp.where(i0 < rows_cached, i0 * ks, OVER_OFF)
                ubase = _vread(u_v, i0)

                def scan_chunk(c, carry2):
                    bestv, bestc = carry2
                    off = pl.multiple_of(c * L, L)
                    cu = (-_chunk_ld(ro + off)
                          - ubase - v_v[pl.ds(off, L)])
                    usedc = used_v[pl.ds(off, L)]
                    unused = usedc < 0.5
                    mv = minv_v[pl.ds(off, L)]
                    upd = unused & (cu < mv)
                    mv2 = jnp.where(upd, cu, mv)
                    minv_v[pl.ds(off, L)] = mv2
                    wc = way_v[pl.ds(off, L)]
                    way_v[pl.ds(off, L)] = jnp.where(upd, jc, wc)
                    elig = unused & (validf_v[pl.ds(off, L)] > 0.5)
                    masked = jnp.where(elig, mv2, BIGF)
                    cmin = jnp.min(masked)
                    better = cmin < bestv
                    return (jnp.where(better, cmin, bestv),
                            jnp.where(better, c, bestc))

                delta, bestc = lax.fori_loop(0, nv, scan_chunk,
                                             (jnp.float32(BIGF), jnp.int32(0)))

                off = pl.multiple_of(bestc * L, L)
                mv2 = minv_v[pl.ds(off, L)]
                elig = ((used_v[pl.ds(off, L)] < 0.5)
                        & (validf_v[pl.ds(off, L)] > 0.5))
                masked = jnp.where(elig, mv2, BIGF)
                lane = _scalar(plsc.all_reduce_ffs(masked <= delta))
                j1 = bestc * L + lane

                def upd_chunk(c, _):
                    off = pl.multiple_of(c * L, L)
                    it = intree_v[pl.ds(off, L)]
                    u_v[pl.ds(off, L)] = u_v[pl.ds(off, L)] + delta * it
                    us = used_v[pl.ds(off, L)]
                    v_v[pl.ds(off, L)] = v_v[pl.ds(off, L)] - delta * us
                    minv_v[pl.ds(off, L)] = (minv_v[pl.ds(off, L)]
                                             - delta * (1.0 - us))
                    return 0

                lax.fori_loop(0, nv, upd_chunk, 0)
                _vwrite(used_v, j1, 1.0)
                i0n = pcol_s[j1]
                return (j1, i0n, (i0n >= 0).astype(jnp.int32))

            jcf, _, _ = lax.while_loop(inner_cond, inner_body,
                                       (jnp.int32(-1), r.astype(jnp.int32),
                                        jnp.int32(1)))

            def aug_body(carry):
                j, _go = carry
                prev = _vread(way_v, j)
                pv = pcol_s[jnp.maximum(prev, 0)]
                pcol_s[j] = jnp.where(prev >= 0, pv, r.astype(jnp.int32))
                return (jnp.maximum(prev, 0), (prev >= 0).astype(jnp.int32))

            lax.while_loop(lambda c: c[1] != 0, aug_body, (jcf, jnp.int32(1)))
            return 0

        lax.fori_loop(0, k, phase, 0)

        def sum_chunk(c, acc):
            off = pl.multiple_of(c * L, L)
            return acc + jnp.sum(u_v[pl.ds(off, L)]) + jnp.sum(v_v[pl.ds(off, L)])

        val = lax.fori_loop(0, nv, sum_chunk, jnp.float32(0.0))
        out_v[...] = jnp.full((L,), val, jnp.float32)

    pltpu.sync_copy(out_v, out_hbm.at[wid])


def _sc_matching(logits_flat, gt, types, mask_i):
    mesh = plsc.VectorSubcoreMesh(core_axis_name="c", subcore_axis_name="s")
    f = pl.kernel(
        _sc_body,
        out_type=jax.ShapeDtypeStruct((NG, L), jnp.float32),
        mesh=mesh,
        compiler_params=pltpu.CompilerParams(needs_layout_passes=False),
        scratch_types=[
            pltpu.VMEM((N,), jnp.int32),
            pltpu.VMEM((N,), jnp.int32),
            pltpu.VMEM((N,), jnp.int32),
            pltpu.VMEM((N,), jnp.int32),
            pltpu.VMEM((N,), jnp.int32),
            pltpu.VMEM((N,), jnp.int32),
            pltpu.VMEM((CAP_WORDS,), jnp.float32),
            pltpu.VMEM((N,), jnp.float32),
            pltpu.VMEM((N,), jnp.float32),
            pltpu.VMEM((N,), jnp.float32),
            pltpu.VMEM((N,), jnp.float32),
            pltpu.VMEM((N,), jnp.int32),
            pltpu.VMEM((N,), jnp.float32),
            pltpu.VMEM((N,), jnp.float32),
            pltpu.VMEM((N,), jnp.float32),
            pltpu.VMEM((512,), jnp.int32),
            pltpu.VMEM((L,), jnp.float32),
            pltpu.SMEM((N,), jnp.int32),
            pltpu.SemaphoreType.DMA,
        ],
    )
    return f(logits_flat, gt, types, mask_i)


def kernel(logits, gt_tokens, node_types, mask_positions):
    logits = logits.astype(jnp.float32)
    gt = gt_tokens.astype(jnp.int32)
    types = node_types.astype(jnp.int32)
    mask_i = mask_positions.astype(jnp.int32)

    lse_sum, cnt = _masked_lse(logits, mask_positions.astype(jnp.float32))
    duals = _sc_matching(logits.reshape(-1), gt, types, mask_i)
    dual_sum = jnp.sum(duals[:, 0])
    loss = (lse_sum + dual_sum) / cnt
    return jnp.where(cnt > 0, loss, jnp.zeros((), jnp.float32))
